# Initial kernel scaffold; baseline (speedup 1.0000x reference)
import jax
import jax.numpy as jnp
from jax.experimental import pallas as pl


def kernel(base_feature, im_info, gt_boxes, W_conv, b_conv, W_cls, b_cls, W_bbox, b_bbox):
    raise NotImplementedError("write your pallas kernel here")


# speedup vs baseline: 43.6305x; 43.6305x over previous
import numpy as np
import jax
import jax.numpy as jnp
from jax.experimental import pallas as pl

_A = 9
_PA = 16
_FH, _FW = 64, 100
_P = _FH * _FW
_STRIDE = 16
_PRE = 6000
_POST = 300
_NMS_TH = 0.7
_MIN_SIZE = 16.0
_NPAD = 6144
_ROWS = 48


def _anchor_params():
    scales = np.array([8.0, 16.0, 32.0])
    ratios = np.array([0.5, 1.0, 2.0])
    size = 16.0 * 16.0
    ws0 = np.round(np.sqrt(size / ratios))
    hs0 = np.round(ws0 * ratios)
    boxes = []
    for w0, h0 in zip(ws0, hs0):
        ws2 = w0 * scales
        hs2 = h0 * scales
        boxes.append(np.stack([7.5 - 0.5 * (ws2 - 1), 7.5 - 0.5 * (hs2 - 1),
                               7.5 + 0.5 * (ws2 - 1), 7.5 + 0.5 * (hs2 - 1)], axis=1))
    anc = np.concatenate(boxes, axis=0).astype(np.float32)
    wa = (anc[:, 2] - anc[:, 0] + np.float32(1.0)).astype(np.float32)
    ha = (anc[:, 3] - anc[:, 1] + np.float32(1.0)).astype(np.float32)
    cxa = (anc[:, 0] + np.float32(0.5) * wa).astype(np.float32)
    cya = (anc[:, 1] + np.float32(0.5) * ha).astype(np.float32)
    out = np.ones((4, _PA), dtype=np.float32)
    out[0, :_A] = wa
    out[1, :_A] = ha
    out[2, :_A] = cxa
    out[3, :_A] = cya
    return out


_ANC = _anchor_params()


def _decode_body(feat_ref, wb_ref, bb_ref, anc_ref, sx_ref, sy_ref, im_ref, out_ref):
    feat = feat_ref[:, :]
    wb = wb_ref[:, :]
    pred = jnp.dot(wb, feat, preferred_element_type=jnp.float32) + bb_ref[:, :]
    dx = pred[0 * _PA:1 * _PA, :]
    dy = pred[1 * _PA:2 * _PA, :]
    dw = pred[2 * _PA:3 * _PA, :]
    dh = pred[3 * _PA:4 * _PA, :]
    wa = anc_ref[0:1, :].reshape(_PA, 1)
    ha = anc_ref[1:2, :].reshape(_PA, 1)
    cxa = anc_ref[2:3, :].reshape(_PA, 1)
    cya = anc_ref[3:4, :].reshape(_PA, 1)
    sx = sx_ref[:, :]
    sy = sy_ref[:, :]
    xmax = im_ref[0, 1] - 1.0
    ymax = im_ref[0, 0] - 1.0
    pcx = dx * wa + (cxa + sx)
    pcy = dy * ha + (cya + sy)
    pw = jnp.exp(dw) * wa
    ph = jnp.exp(dh) * ha
    x1 = jnp.clip(pcx - 0.5 * pw, 0.0, xmax)
    y1 = jnp.clip(pcy - 0.5 * ph, 0.0, ymax)
    x2 = jnp.clip(pcx + 0.5 * pw, 0.0, xmax)
    y2 = jnp.clip(pcy + 0.5 * ph, 0.0, ymax)
    out_ref[0, :, :] = x1
    out_ref[1, :, :] = y1
    out_ref[2, :, :] = x2
    out_ref[3, :, :] = y2


def _nms_body(coords_ref, boxes_ref, supp_ref):
    x1a = coords_ref[0, :, :]
    y1a = coords_ref[1, :, :]
    x2a = coords_ref[2, :, :]
    y2a = coords_ref[3, :, :]
    areas = (x2a - x1a + 1.0) * (y2a - y1a + 1.0)
    idx2d = (jax.lax.broadcasted_iota(jnp.int32, (_ROWS, 128), 0) * 128
             + jax.lax.broadcasted_iota(jnp.int32, (_ROWS, 128), 1))

    def body(i, supp):
        row = boxes_ref[pl.ds(i, 1), :]
        x1i = row[0, 0]
        y1i = row[0, 1]
        x2i = row[0, 2]
        y2i = row[0, 3]
        ar_i = (x2i - x1i + 1.0) * (y2i - y1i + 1.0)
        supp_i = jnp.max(jnp.where(idx2d == i, supp, 0.0))
        iw = jnp.maximum(0.0, jnp.minimum(x2i, x2a) - jnp.maximum(x1i, x1a) + 1.0)
        ih = jnp.maximum(0.0, jnp.minimum(y2i, y2a) - jnp.maximum(y1i, y1a) + 1.0)
        inter = iw * ih
        iou = inter / (ar_i + areas - inter)
        cond = (iou > _NMS_TH) & (idx2d > i) & (supp_i < 0.5)
        return jnp.where(cond, 1.0, supp)

    supp = jax.lax.fori_loop(0, _PRE, body, jnp.zeros((_ROWS, 128), jnp.float32))
    supp_ref[:, :] = supp


def kernel(base_feature, im_info, gt_boxes, W_conv, b_conv, W_cls, b_cls, W_bbox, b_bbox):
    rpn_conv = jax.nn.relu(
        jax.lax.conv_general_dilated(base_feature, W_conv, (1, 1), 'SAME',
                                     dimension_numbers=('NCHW', 'OIHW', 'NCHW'))
        + b_conv[None, :, None, None])
    cls_score = (jax.lax.conv_general_dilated(rpn_conv, W_cls, (1, 1), 'VALID',
                                              dimension_numbers=('NCHW', 'OIHW', 'NCHW'))
                 + b_cls[None, :, None, None])
    B, C, H, W = cls_score.shape
    prob = jax.nn.softmax(cls_score.reshape(B, 2, -1, W), axis=1).reshape(B, C, H, W)
    scores = jnp.transpose(prob[:, _A:, :, :], (0, 2, 3, 1)).reshape(-1)

    feat2d = rpn_conv.reshape(128, _P)
    wb = W_bbox.reshape(_A, 4, 128).transpose(1, 0, 2)
    wb = jnp.concatenate([wb, jnp.zeros((4, _PA - _A, 128), jnp.float32)], axis=1)
    wb = wb.reshape(4 * _PA, 128)
    bb = b_bbox.reshape(_A, 4).T
    bb = jnp.concatenate([bb, jnp.zeros((4, _PA - _A), jnp.float32)], axis=1)
    bb = bb.reshape(4 * _PA, 1)
    px = jnp.arange(_FW, dtype=jnp.float32) * _STRIDE
    py = jnp.arange(_FH, dtype=jnp.float32) * _STRIDE
    sx = jnp.tile(px, _FH).reshape(1, _P)
    sy = jnp.repeat(py, _FW).reshape(1, _P)
    anc = jnp.asarray(_ANC)

    coords = pl.pallas_call(
        _decode_body,
        out_shape=jax.ShapeDtypeStruct((4, _PA, _P), jnp.float32),
    )(feat2d, wb, bb, anc, sx, sy, im_info)

    proposals = coords[:, :_A, :].transpose(2, 1, 0).reshape(-1, 4)

    ws = proposals[:, 2] - proposals[:, 0] + 1.0
    hs = proposals[:, 3] - proposals[:, 1] + 1.0
    min_sz = _MIN_SIZE * im_info[0, 2]
    scores = jnp.where((ws >= min_sz) & (hs >= min_sz), scores, -jnp.inf)
    order = jnp.argsort(-scores)[:_PRE]
    boxes_s = proposals[order]

    boxes_p = jnp.concatenate(
        [boxes_s, jnp.zeros((_NPAD - _PRE, 4), jnp.float32)], axis=0)
    coords4 = boxes_p.T.reshape(4, _ROWS, 128)

    supp = pl.pallas_call(
        _nms_body,
        out_shape=jax.ShapeDtypeStruct((_ROWS, 128), jnp.float32),
    )(coords4, boxes_p)

    supp_flat = supp.reshape(_NPAD)[:_PRE]
    idx = jnp.arange(_PRE)
    pri = jnp.where(supp_flat < 0.5, idx, idx + _PRE)
    keep = jnp.argsort(pri)[:_POST]
    rois_boxes = boxes_s[keep]
    rois = jnp.concatenate(
        [jnp.zeros((_POST, 1), rois_boxes.dtype), rois_boxes], axis=1)[None, :, :]
    return (rois, jnp.float32(0.0), jnp.float32(0.0))
